# baseline (speedup 1.0000x reference)
import functools

import jax
import jax.numpy as jnp
from jax import lax
from jax.experimental import pallas as pl
from jax.experimental.pallas import tpu as pltpu
from jax.experimental.pallas import tpu_sc as plsc

NC = 2
NS = 16
NW = NC * NS

LB = 2048
NRING = 10


def _gz_body(b_per_w, d, v_rows, idx_hbm, tab_hbm, embT_hbm,
             idx_v, slab_v, tail_v, cols_v, sem, tsem):
  cid = lax.axis_index("c")
  sid = lax.axis_index("s")
  wid = sid * NC + cid
  base = wid * b_per_w
  n_full = (v_rows // 128) * 128

  pltpu.sync_copy(idx_hbm.at[pl.ds(base, b_per_w)], idx_v)

  lane16 = lax.iota(jnp.int32, 16)

  def idx_at(e):
    v = idx_v[pl.ds((e // 16) * 16, 16)]
    return jnp.max(jnp.where(lane16 == e % 16, v, jnp.int32(-1)))

  pltpu.async_copy(tab_hbm.at[:, pl.ds(n_full, v_rows - n_full)],
                   tail_v, tsem).wait()

  jvs = [lax.iota(jnp.int32, 16) + 16 * k for k in range(d // 16)]

  def fire(e):
    i = idx_at(e)
    c = jnp.minimum(i, n_full - 1) // 128
    pltpu.async_copy(tab_hbm.at[:, pl.ds(c * 128, 128)],
                     slab_v.at[e % NRING], sem)

  def extract(e):
    i = idx_at(e)
    l = i % 128
    ev = jnp.full((16,), e, jnp.int32)

    @pl.when(i < n_full)
    def _():
      lv = jnp.full((16,), l, jnp.int32)
      for k in range(d // 16):
        v = plsc.load_gather(slab_v.at[e % NRING], [jvs[k], lv])
        plsc.store_scatter(cols_v, [jvs[k], ev], v)

    @pl.when(i >= n_full)
    def _():
      lv = jnp.full((16,), i - n_full, jnp.int32)
      for k in range(d // 16):
        v = plsc.load_gather(tail_v, [jvs[k], lv])
        plsc.store_scatter(cols_v, [jvs[k], ev], v)

  for e in range(NRING - 1):
    fire(e)

  def step(e, carry):
    pltpu.make_async_copy(tab_hbm.at[:, pl.ds(0, 128)],
                          slab_v.at[0], sem).wait()
    fire(e + NRING - 1)
    extract(e)
    return carry
  lax.fori_loop(0, b_per_w - (NRING - 1), step, 0)

  for e in range(b_per_w - (NRING - 1), b_per_w):
    pltpu.make_async_copy(tab_hbm.at[:, pl.ds(0, 128)],
                          slab_v.at[0], sem).wait()
    extract(e)

  pltpu.sync_copy(cols_v, embT_hbm.at[:, pl.ds(base, b_per_w)])


def _t1_body(ctx_ref, out_ref):
  s = pl.program_id(0)
  n = ctx_ref.shape[1]
  oh = (lax.broadcasted_iota(jnp.int32, (n, 1), 0) == s).astype(jnp.float32)
  col = jnp.dot(ctx_ref[...], oh, preferred_element_type=jnp.float32,
                precision=lax.Precision.HIGHEST)
  out_ref[...] = jnp.broadcast_to(col, out_ref.shape[1:])[None]


def _t2_body(_, embT_ref, out_ref):
  out_ref[...] = embT_ref[...][None]


def kernel(class_indices, ctx, class_embeddings):
  B = class_indices.shape[0]
  n_ctx, d = ctx.shape
  b_per_w = B // NW

  idx = class_indices.astype(jnp.int32)
  tab_t = class_embeddings.T
  v_rows = class_embeddings.shape[0]
  n_tail = v_rows - (v_rows // 128) * 128

  mesh = plsc.VectorSubcoreMesh(core_axis_name="c", subcore_axis_name="s")
  emb_t = pl.kernel(
      functools.partial(_gz_body, b_per_w, d, v_rows),
      out_type=jax.ShapeDtypeStruct((d, B), jnp.float32),
      mesh=mesh,
      compiler_params=pltpu.CompilerParams(needs_layout_passes=False),
      scratch_types=[
          pltpu.VMEM((b_per_w,), jnp.int32),
          pltpu.VMEM((NRING, d, 128), jnp.float32),
          pltpu.VMEM((d, n_tail), jnp.float32),
          pltpu.VMEM((d, b_per_w), jnp.float32),
          pltpu.SemaphoreType.DMA,
          pltpu.SemaphoreType.DMA,
      ],
  )(idx, tab_t)

  ctx_t = ctx.T
  out17 = pl.pallas_call(
      _t1_body,
      grid=(n_ctx, B // LB),
      in_specs=[pl.BlockSpec((d, n_ctx), lambda s, t: (0, 0))],
      out_specs=pl.BlockSpec((1, d, LB), lambda s, t: (s, 0, t)),
      out_shape=jax.ShapeDtypeStruct((n_ctx + 1, d, B), jnp.float32),
  )(ctx_t)

  out17 = pl.pallas_call(
      _t2_body,
      grid=(B // LB,),
      in_specs=[
          pl.BlockSpec(memory_space=pltpu.MemorySpace.HBM),
          pl.BlockSpec((d, LB), lambda t: (0, t)),
      ],
      out_specs=pl.BlockSpec((1, d, LB), lambda t: (n_ctx, 0, t)),
      out_shape=jax.ShapeDtypeStruct((n_ctx + 1, d, B), jnp.float32),
      input_output_aliases={0: 0},
  )(out17, emb_t)

  return out17.transpose(2, 0, 1)


# speedup vs baseline: 1.0325x; 1.0325x over previous
import functools

import jax
import jax.numpy as jnp
from jax import lax
from jax.experimental import pallas as pl
from jax.experimental.pallas import tpu as pltpu
from jax.experimental.pallas import tpu_sc as plsc

NC = 2
NS = 16
NW = NC * NS

LB = 2048
NRING = 8


def _gz_body(b_per_w, d, v_rows, idx_hbm, tab_hbm, embT_hbm,
             idx_v, slab_v, tail_v, cols_v, sem, tsem):
  cid = lax.axis_index("c")
  sid = lax.axis_index("s")
  wid = sid * NC + cid
  base = wid * b_per_w
  n_full = (v_rows // 128) * 128

  pltpu.sync_copy(idx_hbm.at[pl.ds(base, b_per_w)], idx_v)

  lane16 = lax.iota(jnp.int32, 16)

  def idx_at(e):
    v = idx_v[pl.ds((e // 16) * 16, 16)]
    return jnp.max(jnp.where(lane16 == e % 16, v, jnp.int32(-1)))

  pltpu.async_copy(tab_hbm.at[:, pl.ds(n_full, v_rows - n_full)],
                   tail_v, tsem).wait()

  jvs = [lax.iota(jnp.int32, 16) + 16 * k for k in range(d // 16)]

  def fire(e):
    i = idx_at(e)
    c = jnp.minimum(i, n_full - 1) // 128
    pltpu.async_copy(tab_hbm.at[:, pl.ds(c * 128, 128)],
                     slab_v.at[e % NRING], sem)

  def extract(e):
    i = idx_at(e)
    l = i % 128
    ev = jnp.full((16,), e, jnp.int32)

    @pl.when(i < n_full)
    def _():
      lv = jnp.full((16,), l, jnp.int32)
      for k in range(d // 16):
        v = plsc.load_gather(slab_v.at[e % NRING], [jvs[k], lv])
        plsc.store_scatter(cols_v, [jvs[k], ev], v)

    @pl.when(i >= n_full)
    def _():
      lv = jnp.full((16,), i - n_full, jnp.int32)
      for k in range(d // 16):
        v = plsc.load_gather(tail_v, [jvs[k], lv])
        plsc.store_scatter(cols_v, [jvs[k], ev], v)

  NP = NRING - 2
  for e in range(NP):
    fire(e)

  def step(g, carry):
    for u in range(2):
      e = g * 2 + u
      pltpu.make_async_copy(tab_hbm.at[:, pl.ds(0, 128)],
                            slab_v.at[0], sem).wait()
      fire(e + NP)
      extract(e)
    return carry
  n_main = b_per_w - NP
  assert n_main % 2 == 0
  lax.fori_loop(0, n_main // 2, step, 0)

  for e in range(b_per_w - NP, b_per_w):
    pltpu.make_async_copy(tab_hbm.at[:, pl.ds(0, 128)],
                          slab_v.at[0], sem).wait()
    extract(e)

  pltpu.sync_copy(cols_v, embT_hbm.at[:, pl.ds(base, b_per_w)])


def _t1_body(ctx_ref, out_ref):
  s = pl.program_id(0)
  n = ctx_ref.shape[1]
  oh = (lax.broadcasted_iota(jnp.int32, (n, 1), 0) == s).astype(jnp.float32)
  col = jnp.dot(ctx_ref[...], oh, preferred_element_type=jnp.float32,
                precision=lax.Precision.HIGHEST)
  out_ref[...] = jnp.broadcast_to(col, out_ref.shape[1:])[None]


def _t2_body(_, embT_ref, out_ref):
  out_ref[...] = embT_ref[...][None]


def kernel(class_indices, ctx, class_embeddings):
  B = class_indices.shape[0]
  n_ctx, d = ctx.shape
  b_per_w = B // NW

  idx = class_indices.astype(jnp.int32)
  tab_t = class_embeddings.T
  v_rows = class_embeddings.shape[0]
  n_tail = v_rows - (v_rows // 128) * 128

  mesh = plsc.VectorSubcoreMesh(core_axis_name="c", subcore_axis_name="s")
  emb_t = pl.kernel(
      functools.partial(_gz_body, b_per_w, d, v_rows),
      out_type=jax.ShapeDtypeStruct((d, B), jnp.float32),
      mesh=mesh,
      compiler_params=pltpu.CompilerParams(needs_layout_passes=False),
      scratch_types=[
          pltpu.VMEM((b_per_w,), jnp.int32),
          pltpu.VMEM((NRING, d, 128), jnp.float32),
          pltpu.VMEM((d, n_tail), jnp.float32),
          pltpu.VMEM((d, b_per_w), jnp.float32),
          pltpu.SemaphoreType.DMA,
          pltpu.SemaphoreType.DMA,
      ],
  )(idx, tab_t)

  ctx_t = ctx.T
  out17 = pl.pallas_call(
      _t1_body,
      grid=(n_ctx, B // LB),
      in_specs=[pl.BlockSpec((d, n_ctx), lambda s, t: (0, 0))],
      out_specs=pl.BlockSpec((1, d, LB), lambda s, t: (s, 0, t)),
      out_shape=jax.ShapeDtypeStruct((n_ctx + 1, d, B), jnp.float32),
  )(ctx_t)

  out17 = pl.pallas_call(
      _t2_body,
      grid=(B // LB,),
      in_specs=[
          pl.BlockSpec(memory_space=pltpu.MemorySpace.HBM),
          pl.BlockSpec((d, LB), lambda t: (0, t)),
      ],
      out_specs=pl.BlockSpec((1, d, LB), lambda t: (n_ctx, 0, t)),
      out_shape=jax.ShapeDtypeStruct((n_ctx + 1, d, B), jnp.float32),
      input_output_aliases={0: 0},
  )(out17, emb_t)

  return out17.transpose(2, 0, 1)
